# Initial kernel scaffold; baseline (speedup 1.0000x reference)
import functools

import jax
import jax.numpy as jnp
from jax import lax
from jax.experimental import pallas as pl
from jax.experimental.pallas import tpu as pltpu
from jax.experimental.pallas import tpu_sc as plsc

VOCAB = 1000000
EMBED_DIM = 64
OUT_DIM = 64
BATCH = 16384
HIST = 50
B_TOTAL = BATCH * HIST

_ROWS_PER_BLOCK = 4000


def _transform_body(x_ref, wt_ref, b_ref, o_ref):
    acc = jnp.dot(x_ref[...], wt_ref[...], preferred_element_type=jnp.float32)
    o_ref[...] = jnp.maximum(acc + b_ref[...], 0.0)


def _transform_table(table, wt, b2):
    grid = (VOCAB // _ROWS_PER_BLOCK,)
    return pl.pallas_call(
        _transform_body,
        grid=grid,
        in_specs=[
            pl.BlockSpec((_ROWS_PER_BLOCK, EMBED_DIM), lambda i: (i, 0)),
            pl.BlockSpec((EMBED_DIM, OUT_DIM), lambda i: (0, 0)),
            pl.BlockSpec((1, OUT_DIM), lambda i: (0, 0)),
        ],
        out_specs=pl.BlockSpec((_ROWS_PER_BLOCK, OUT_DIM), lambda i: (i, 0)),
        out_shape=jax.ShapeDtypeStruct((VOCAB, OUT_DIM), jnp.float32),
    )(table, wt, b2)


_info = plsc.get_sparse_core_info()
_NC, _NS = _info.num_cores, _info.num_subcores
_NW = _NC * _NS
_B_PER_W = B_TOTAL // _NW
_IDXW = 128
_ROWS_PER_CHUNK = 512
_STREAMS_PER_CHUNK = _ROWS_PER_CHUNK // _IDXW
_CHUNKS = _B_PER_W // _ROWS_PER_CHUNK
_IDX_ROWS_PER_W = _B_PER_W // _IDXW

_mesh = plsc.VectorSubcoreMesh(core_axis_name="c", subcore_axis_name="s")


@functools.partial(
    pl.kernel,
    mesh=_mesh,
    out_type=jax.ShapeDtypeStruct((B_TOTAL, OUT_DIM), jnp.float32),
    scratch_types=[
        pltpu.VMEM((_IDX_ROWS_PER_W, _IDXW), jnp.int32),
        pltpu.VMEM((_ROWS_PER_CHUNK, OUT_DIM), jnp.float32),
        pltpu.SemaphoreType.DMA,
    ],
)
def _gather_rows(idx_hbm, t2_hbm, out_hbm, idx_v, rows_v, sem):
    wid = lax.axis_index("s") * _NC + lax.axis_index("c")
    base = wid * _B_PER_W
    pltpu.sync_copy(idx_hbm.at[pl.ds(wid * _IDX_ROWS_PER_W, _IDX_ROWS_PER_W)], idx_v)

    def chunk_body(c, carry):
        cps = []
        for k in range(_STREAMS_PER_CHUNK):
            cps.append(
                pltpu.async_copy(
                    t2_hbm.at[idx_v.at[c * _STREAMS_PER_CHUNK + k]],
                    rows_v.at[pl.ds(k * _IDXW, _IDXW)],
                    sem,
                )
            )
        for cp in cps:
            cp.wait()
        pltpu.sync_copy(rows_v, out_hbm.at[pl.ds(base + c * _ROWS_PER_CHUNK, _ROWS_PER_CHUNK)])
        return carry

    lax.fori_loop(0, _CHUNKS, chunk_body, 0)


def kernel(element, table, W, b):
    idx = element.reshape(-1).astype(jnp.int32).reshape(B_TOTAL // _IDXW, _IDXW)
    t2 = _transform_table(table, W.T, b.reshape(1, OUT_DIM))
    out = _gather_rows(idx, t2)
    return out.reshape(BATCH, HIST, OUT_DIM)


# speedup vs baseline: 1.2842x; 1.2842x over previous
import functools

import jax
import jax.numpy as jnp
from jax import lax
from jax.experimental import pallas as pl
from jax.experimental.pallas import tpu as pltpu
from jax.experimental.pallas import tpu_sc as plsc

VOCAB = 1000000
EMBED_DIM = 64
OUT_DIM = 64
BATCH = 16384
HIST = 50
B_TOTAL = BATCH * HIST

_ROWS_PER_BLOCK = 4000


def _transform_body(x_ref, wt_ref, b_ref, o_ref):
    acc = jnp.dot(x_ref[...], wt_ref[...], preferred_element_type=jnp.float32)
    o_ref[...] = jnp.maximum(acc + b_ref[...], 0.0)


def _transform_table(table, wt, b2):
    grid = (VOCAB // _ROWS_PER_BLOCK,)
    return pl.pallas_call(
        _transform_body,
        grid=grid,
        in_specs=[
            pl.BlockSpec((_ROWS_PER_BLOCK, EMBED_DIM), lambda i: (i, 0)),
            pl.BlockSpec((EMBED_DIM, OUT_DIM), lambda i: (0, 0)),
            pl.BlockSpec((1, OUT_DIM), lambda i: (0, 0)),
        ],
        out_specs=pl.BlockSpec((_ROWS_PER_BLOCK, OUT_DIM), lambda i: (i, 0)),
        out_shape=jax.ShapeDtypeStruct((VOCAB, OUT_DIM), jnp.float32),
    )(table, wt, b2)


_info = plsc.get_sparse_core_info()
_NC, _NS = _info.num_cores, _info.num_subcores
_NW = _NC * _NS
_B_PER_W = B_TOTAL // _NW
_IDXW = 128
_ROWS_PER_CHUNK = 512
_STREAMS_PER_CHUNK = _ROWS_PER_CHUNK // _IDXW
_CHUNKS = _B_PER_W // _ROWS_PER_CHUNK
_IDX_ROWS_PER_W = _B_PER_W // _IDXW

_mesh = plsc.VectorSubcoreMesh(core_axis_name="c", subcore_axis_name="s")


@functools.partial(
    pl.kernel,
    mesh=_mesh,
    compiler_params=pltpu.CompilerParams(use_tc_tiling_on_sc=False),
    out_type=jax.ShapeDtypeStruct((B_TOTAL, OUT_DIM), jnp.float32),
    scratch_types=[
        pltpu.VMEM((_IDX_ROWS_PER_W, _IDXW), jnp.int32),
        pltpu.VMEM((_ROWS_PER_CHUNK, OUT_DIM), jnp.float32),
        pltpu.SemaphoreType.DMA,
    ],
)
def _gather_rows(idx_hbm, t2_hbm, out_hbm, idx_v, rows_v, sem):
    wid = lax.axis_index("s") * _NC + lax.axis_index("c")
    base = wid * _B_PER_W
    pltpu.sync_copy(idx_hbm.at[pl.ds(wid * _IDX_ROWS_PER_W, _IDX_ROWS_PER_W)], idx_v)

    def chunk_body(c, carry):
        cps = []
        for k in range(_STREAMS_PER_CHUNK):
            cps.append(
                pltpu.async_copy(
                    t2_hbm.at[idx_v.at[c * _STREAMS_PER_CHUNK + k]],
                    rows_v.at[pl.ds(k * _IDXW, _IDXW)],
                    sem,
                )
            )
        for cp in cps:
            cp.wait()
        pltpu.sync_copy(rows_v, out_hbm.at[pl.ds(base + c * _ROWS_PER_CHUNK, _ROWS_PER_CHUNK)])
        return carry

    lax.fori_loop(0, _CHUNKS, chunk_body, 0)


def kernel(element, table, W, b):
    idx = element.reshape(-1).astype(jnp.int32).reshape(B_TOTAL // _IDXW, _IDXW)
    t2 = _transform_table(table, W.T, b.reshape(1, OUT_DIM))
    out = _gather_rows(idx, t2)
    return out.reshape(BATCH, HIST, OUT_DIM)


# speedup vs baseline: 2.5834x; 2.0117x over previous
import functools

import jax
import jax.numpy as jnp
from jax import lax
from jax.experimental import pallas as pl
from jax.experimental.pallas import tpu as pltpu
from jax.experimental.pallas import tpu_sc as plsc

VOCAB = 1000000
EMBED_DIM = 64
OUT_DIM = 64
BATCH = 16384
HIST = 50
B_TOTAL = BATCH * HIST
HALF = BATCH // 2

_CB = 4096
_NBLK = (VOCAB + _CB - 1) // _CB
_VPAD = _NBLK * _CB


def _transform_body(xt_ref, wt_ref, b_ref, o_ref):
    acc = lax.dot_general(
        xt_ref[...],
        wt_ref[...],
        (((0,), (0,)), ((), ())),
        preferred_element_type=jnp.float32,
    )
    z = jnp.maximum(acc + b_ref[...], 0.0)
    o_ref[:, 0:OUT_DIM] = z[0 : _CB // 2]
    o_ref[:, OUT_DIM:128] = z[_CB // 2 : _CB]


def _transform_table(table_t, wt, b2):
    grid = (_NBLK,)
    return pl.pallas_call(
        _transform_body,
        grid=grid,
        in_specs=[
            pl.BlockSpec((EMBED_DIM, _CB), lambda i: (0, i)),
            pl.BlockSpec((EMBED_DIM, OUT_DIM), lambda i: (0, 0)),
            pl.BlockSpec((1, OUT_DIM), lambda i: (0, 0)),
        ],
        out_specs=pl.BlockSpec((_CB // 2, 128), lambda i: (i, 0)),
        out_shape=jax.ShapeDtypeStruct((_VPAD // 2, 128), jnp.float32),
    )(table_t, wt, b2)


_info = plsc.get_sparse_core_info()
_NC, _NS = _info.num_cores, _info.num_subcores
_NW = _NC * _NS
_B_PER_W = B_TOTAL // _NW
_IDXW = 128
_ROWS_PER_CHUNK = 512
_STREAMS_PER_CHUNK = _ROWS_PER_CHUNK // _IDXW
_CHUNKS = _B_PER_W // _ROWS_PER_CHUNK
_IDX_ROWS_PER_W = _B_PER_W // _IDXW

_mesh = plsc.VectorSubcoreMesh(core_axis_name="c", subcore_axis_name="s")


@functools.partial(
    pl.kernel,
    mesh=_mesh,
    compiler_params=pltpu.CompilerParams(use_tc_tiling_on_sc=False),
    out_type=jax.ShapeDtypeStruct((B_TOTAL, OUT_DIM), jnp.float32),
    scratch_types=[
        pltpu.VMEM((_IDX_ROWS_PER_W, _IDXW), jnp.int32),
        pltpu.VMEM((_ROWS_PER_CHUNK, OUT_DIM), jnp.float32),
        pltpu.SemaphoreType.DMA,
    ],
)
def _gather_rows(idx_hbm, t2_hbm, out_hbm, idx_v, rows_v, sem):
    wid = lax.axis_index("s") * _NC + lax.axis_index("c")
    base = wid * _B_PER_W
    pltpu.sync_copy(idx_hbm.at[pl.ds(wid * _IDX_ROWS_PER_W, _IDX_ROWS_PER_W)], idx_v)

    def chunk_body(c, carry):
        cps = []
        for k in range(_STREAMS_PER_CHUNK):
            cps.append(
                pltpu.async_copy(
                    t2_hbm.at[idx_v.at[c * _STREAMS_PER_CHUNK + k]],
                    rows_v.at[pl.ds(k * _IDXW, _IDXW)],
                    sem,
                )
            )
        for cp in cps:
            cp.wait()
        pltpu.sync_copy(rows_v, out_hbm.at[pl.ds(base + c * _ROWS_PER_CHUNK, _ROWS_PER_CHUNK)])
        return carry

    lax.fori_loop(0, _CHUNKS, chunk_body, 0)


def _transpose_body(x_ref, o_ref):
    xt = x_ref[...].T
    o_ref[0, :, 0:HALF] = xt[0:OUT_DIM, :]
    o_ref[0, :, HALF:BATCH] = xt[OUT_DIM:128, :]


def _transpose_out(out2_pairs):
    grid = (HIST,)
    return pl.pallas_call(
        _transpose_body,
        grid=grid,
        in_specs=[pl.BlockSpec((HALF, 128), lambda l: (l, 0))],
        out_specs=pl.BlockSpec((1, OUT_DIM, BATCH), lambda l: (l, 0, 0)),
        out_shape=jax.ShapeDtypeStruct((HIST, OUT_DIM, BATCH), jnp.float32),
    )(out2_pairs)


def kernel(element, table, W, b):
    el = element.astype(jnp.int32)
    elp = el.T.reshape(HIST, 2, HALF).transpose(0, 2, 1).reshape(HIST, BATCH)
    c = elp & (_CB - 1)
    g = (elp & ~(_CB - 1)) + 2 * (c & (_CB // 2 - 1)) + (c >> 11)
    idx = g.reshape(B_TOTAL // _IDXW, _IDXW)

    t2c = _transform_table(table.T, W.T, b.reshape(1, OUT_DIM))
    t2 = t2c.reshape(_VPAD, OUT_DIM)
    out2 = _gather_rows(idx, t2)
    a = _transpose_out(out2.reshape(B_TOTAL // 2, 128))
    return jnp.transpose(a, (2, 0, 1))


# speedup vs baseline: 3.2935x; 1.2749x over previous
import functools

import jax
import jax.numpy as jnp
from jax import lax
from jax.experimental import pallas as pl
from jax.experimental.pallas import tpu as pltpu
from jax.experimental.pallas import tpu_sc as plsc

VOCAB = 1000000
EMBED_DIM = 64
OUT_DIM = 64
BATCH = 16384
HIST = 50
B_TOTAL = BATCH * HIST
QUARTER = BATCH // 4
WPR = OUT_DIM // 2

_CB = 4096
_NBLK = (VOCAB + _CB - 1) // _CB
_VPAD = _NBLK * _CB


def _pack_bf16(z):
    lo = lax.convert_element_type(
        lax.bitcast_convert_type(z[:, 0:WPR].astype(jnp.bfloat16), jnp.uint16),
        jnp.uint32,
    )
    hi = lax.convert_element_type(
        lax.bitcast_convert_type(z[:, WPR : 2 * WPR].astype(jnp.bfloat16), jnp.uint16),
        jnp.uint32,
    )
    return lax.bitcast_convert_type(lo | (hi << 16), jnp.float32)


def _transform_body(xt_ref, wt_ref, b_ref, o_ref):
    acc = lax.dot_general(
        xt_ref[...],
        wt_ref[...],
        (((0,), (0,)), ((), ())),
        preferred_element_type=jnp.float32,
    )
    z = jnp.maximum(acc + b_ref[...], 0.0)
    for m in range(4):
        o_ref[:, m * WPR : (m + 1) * WPR] = _pack_bf16(
            z[m * (_CB // 4) : (m + 1) * (_CB // 4)]
        )


def _transform_table(table_t, wt, b2):
    return pl.pallas_call(
        _transform_body,
        grid=(_NBLK,),
        in_specs=[
            pl.BlockSpec((EMBED_DIM, _CB), lambda i: (0, i)),
            pl.BlockSpec((EMBED_DIM, OUT_DIM), lambda i: (0, 0)),
            pl.BlockSpec((1, OUT_DIM), lambda i: (0, 0)),
        ],
        out_specs=pl.BlockSpec((_CB // 4, 128), lambda i: (i, 0)),
        out_shape=jax.ShapeDtypeStruct((_VPAD // 4, 128), jnp.float32),
    )(table_t, wt, b2)


_info = plsc.get_sparse_core_info()
_NC, _NS = _info.num_cores, _info.num_subcores
_NW = _NC * _NS
_B_PER_W = B_TOTAL // _NW
_IDXW = 128
_ROWS_PER_CHUNK = 1024
_STREAMS_PER_CHUNK = _ROWS_PER_CHUNK // _IDXW
_CHUNKS = _B_PER_W // _ROWS_PER_CHUNK
_IDX_ROWS_PER_W = _B_PER_W // _IDXW

_mesh = plsc.VectorSubcoreMesh(core_axis_name="c", subcore_axis_name="s")


@functools.partial(
    pl.kernel,
    mesh=_mesh,
    compiler_params=pltpu.CompilerParams(use_tc_tiling_on_sc=False),
    out_type=jax.ShapeDtypeStruct((B_TOTAL, WPR), jnp.float32),
    scratch_types=[
        pltpu.VMEM((_IDX_ROWS_PER_W, _IDXW), jnp.int32),
        pltpu.VMEM((_ROWS_PER_CHUNK, WPR), jnp.float32),
        pltpu.SemaphoreType.DMA,
    ],
)
def _gather_rows(idx_hbm, t2_hbm, out_hbm, idx_v, rows_v, sem):
    wid = lax.axis_index("s") * _NC + lax.axis_index("c")
    base = wid * _B_PER_W
    pltpu.sync_copy(idx_hbm.at[pl.ds(wid * _IDX_ROWS_PER_W, _IDX_ROWS_PER_W)], idx_v)

    def chunk_body(c, carry):
        cps = []
        for k in range(_STREAMS_PER_CHUNK):
            cps.append(
                pltpu.async_copy(
                    t2_hbm.at[idx_v.at[c * _STREAMS_PER_CHUNK + k]],
                    rows_v.at[pl.ds(k * _IDXW, _IDXW)],
                    sem,
                )
            )
        for cp in cps:
            cp.wait()
        pltpu.sync_copy(rows_v, out_hbm.at[pl.ds(base + c * _ROWS_PER_CHUNK, _ROWS_PER_CHUNK)])
        return carry

    lax.fori_loop(0, _CHUNKS, chunk_body, 0)


def _transpose_body(x_ref, o_ref):
    xt = lax.bitcast_convert_type(x_ref[...], jnp.uint32).T
    for m in range(4):
        w = xt[m * WPR : (m + 1) * WPR, :]
        lo = lax.bitcast_convert_type(w << 16, jnp.float32)
        hi = lax.bitcast_convert_type(w & jnp.uint32(0xFFFF0000), jnp.float32)
        o_ref[0, 0:WPR, m * QUARTER : (m + 1) * QUARTER] = lo
        o_ref[0, WPR : 2 * WPR, m * QUARTER : (m + 1) * QUARTER] = hi


def _transpose_out(out2_words):
    return pl.pallas_call(
        _transpose_body,
        grid=(HIST,),
        in_specs=[pl.BlockSpec((QUARTER, 128), lambda l: (l, 0))],
        out_specs=pl.BlockSpec((1, OUT_DIM, BATCH), lambda l: (l, 0, 0)),
        out_shape=jax.ShapeDtypeStruct((HIST, OUT_DIM, BATCH), jnp.float32),
    )(out2_words)


def kernel(element, table, W, b):
    el = element.astype(jnp.int32)
    elp = el.T.reshape(HIST, 4, QUARTER).transpose(0, 2, 1).reshape(HIST, BATCH)
    c = elp & (_CB - 1)
    g = (elp & ~(_CB - 1)) + 4 * (c & (_CB // 4 - 1)) + (c >> 10)
    idx = g.reshape(B_TOTAL // _IDXW, _IDXW)

    t2c = _transform_table(table.T, W.T, b.reshape(1, OUT_DIM))
    t2 = t2c.reshape(_VPAD, WPR)
    out2 = _gather_rows(idx, t2)
    a = _transpose_out(out2.reshape(B_TOTAL * WPR // 128, 128))
    return jnp.transpose(a, (2, 0, 1))
